# baseline (speedup 1.0000x reference)
import functools

import jax
import jax.numpy as jnp
import numpy as np
from jax import lax
from jax.experimental import pallas as pl
from jax.experimental.pallas import tpu as pltpu
from jax.experimental.pallas import tpu_sc as plsc

B = 1024
D = 64
NUM_GROUPS = 32
G = 8
HID = (D + 2 * D) // 2
Q_STD = float(np.sqrt(2.0 / D))
LOG_SIGMA = float(np.log(Q_STD))

_NW = 32
_BPW = B // _NW


def _sc_gather_body(idx_hbm, table_hbm, out_hbm, idx_v, rows_v, sem):
    wid = lax.axis_index("s") * 2 + lax.axis_index("c")
    base = wid * _BPW
    pltpu.sync_copy(idx_hbm.at[pl.ds(base, _BPW)], idx_v)
    for i in range(_BPW // 16):
        sl = pl.ds(i * 16, 16)
        idx_v[sl] = lax.shift_right_logical(idx_v[sl], 1)
    pltpu.async_copy(table_hbm.at[idx_v], rows_v, sem).wait()
    pltpu.sync_copy(rows_v, out_hbm.at[pl.ds(base, _BPW)])


@jax.jit
def _sc_gather(idx, table2):
    mesh = plsc.VectorSubcoreMesh(core_axis_name="c", subcore_axis_name="s")
    return pl.kernel(
        _sc_gather_body,
        mesh=mesh,
        out_type=jax.ShapeDtypeStruct((B, 2 * D), jnp.float32),
        scratch_types=[
            pltpu.VMEM((_BPW,), jnp.int32),
            pltpu.VMEM((_BPW, 2 * D), jnp.float32),
            pltpu.SemaphoreType.DMA,
        ],
    )(idx, table2)


def _sigmoid(x):
    return 1.0 / (1.0 + jnp.exp(-x))


def _dot(a, b):
    return jax.lax.dot(a, b, precision=jax.lax.Precision.HIGHEST,
                       preferred_element_type=jnp.float32)


def _tc_body(gi_ref, ii_ref, pair_ref, std_ref, u_ref, gemb_ref,
             w1t_ref, b1_ref, w2mut_ref, w2sgt_ref, b2mu_ref, b2sg_ref,
             a1mt_ref, a1it_ref, a1_ref, a2t_ref, a2_ref,
             p1at_ref, p1bt_ref, p1ct_ref, p1_ref, p2t_ref, p2_ref,
             y_ref, y2_ref, dkl_ref):
    relu = lambda x: jnp.maximum(x, 0.0)

    U = [u_ref[j] for j in range(G)]
    gsum = U[0]
    for j in range(1, G):
        gsum = gsum + U[j]
    group_z = relu(gsum * (1.0 / G))
    h = relu(_dot(group_z, w1t_ref[...]) + b1_ref[...])
    z_mu = _dot(h, w2mut_ref[...]) + b2mu_ref[...]
    z_sg = 0.1 + 0.9 * _sigmoid(_dot(h, w2sgt_ref[...]) + b2sg_ref[...])

    pair = pair_ref[...]
    odd = (ii_ref[...] & 1) == 1
    item = jnp.where(odd, pair[:, D:], pair[:, :D])
    ipart = _dot(item, a1it_ref[...]) + a1_ref[...]

    gi = gi_ref[...]
    gids = lax.broadcasted_iota(jnp.int32, (B, NUM_GROUPS), 1)
    onehot = (gi == gids).astype(jnp.float32)

    att_cols = []
    for j in range(G):
        mp = _dot(U[j], a1mt_ref[...])
        hj = relu(_dot(onehot, mp) + ipart)
        att_cols.append(_dot(hj, a2t_ref[...]))
    att = jnp.concatenate(att_cols, axis=1) + a2_ref[...]
    amax = jnp.max(att, axis=1, keepdims=True)
    aexp = jnp.exp(att - amax)
    w = aexp / jnp.sum(aexp, axis=1, keepdims=True)

    g_att = w[:, 0:1] * _dot(onehot, U[0])
    for j in range(1, G):
        g_att = g_att + w[:, j:j + 1] * _dot(onehot, U[j])

    gp = _dot(onehot, gemb_ref[...])
    zmu_b = _dot(onehot, z_mu)
    zsg_b = _dot(onehot, z_sg)
    std = std_ref[...]
    ge = g_att + gp + Q_STD * std
    ge2 = g_att + zmu_b + zsg_b * std

    def ncf(x):
        h1 = relu(_dot(x * item, p1at_ref[...]) + _dot(x, p1bt_ref[...])
                  + _dot(item, p1ct_ref[...]) + p1_ref[...])
        return _sigmoid(_dot(h1, p2t_ref[...]) + p2_ref[...])

    y_ref[...] = ncf(ge)
    y2_ref[...] = ncf(ge2)

    gvec = gemb_ref[...]
    t = (2.0 * jnp.log(z_sg) - 2.0 * LOG_SIGMA
         + (Q_STD * Q_STD) / (z_sg * z_sg)
         + ((gvec - z_mu) * (gvec - z_mu)) / (z_sg * z_sg) - 1.0)
    s = jnp.sum(0.5 * t, axis=1, keepdims=True)
    counts = jnp.sum(onehot, axis=0, keepdims=True)
    dkl_ref[...] = _dot(counts, s) * (1.0 / B)


@functools.partial(jax.jit, static_argnums=())
def _tc_call(gi, ii2, pair_rows, std, u8, group_emb,
             w1t, b1, w2mut, w2sgt, b2mu, b2sg,
             a1mt, a1it, a1, a2t, a2, p1at, p1bt, p1ct, p1, p2t, p2):
    return pl.pallas_call(
        _tc_body,
        out_shape=(
            jax.ShapeDtypeStruct((B, 1), jnp.float32),
            jax.ShapeDtypeStruct((B, 1), jnp.float32),
            jax.ShapeDtypeStruct((1, 1), jnp.float32),
        ),
    )(gi, ii2, pair_rows, std, u8, group_emb,
      w1t, b1, w2mut, w2sgt, b2mu, b2sg,
      a1mt, a1it, a1, a2t, a2, p1at, p1bt, p1ct, p1, p2t, p2)


def kernel(group_inputs, item_inputs, is_training, user_emb, item_emb,
           group_emb, W1, b1, W2, b2, A1, a1, A2, a2, P1, p1, P2, p2):
    gi = group_inputs.astype(jnp.int32).reshape(B, 1)
    ii = item_inputs.astype(jnp.int32)

    table2 = item_emb.reshape(-1, 2 * D)
    pair_rows = _sc_gather(ii, table2)

    u8 = user_emb[:NUM_GROUPS * G].reshape(NUM_GROUPS, G, D).swapaxes(0, 1)
    std = jax.random.normal(jax.random.key(1), (B, D), dtype=jnp.float32)

    w1t = W1.T
    w2mut = W2[:D].T
    w2sgt = W2[D:].T
    a1mt = A1[:, :D].T
    a1it = A1[:, D:].T
    p1at = P1[:, :D].T
    p1bt = P1[:, D:2 * D].T
    p1ct = P1[:, 2 * D:].T

    y, y2, dkl = _tc_call(
        gi, ii.reshape(B, 1), pair_rows, std, u8, group_emb,
        w1t, b1.reshape(1, HID), w2mut, w2sgt,
        b2[:D].reshape(1, D), b2[D:].reshape(1, D),
        a1mt, a1it, a1.reshape(1, 16), A2.T, a2.reshape(1, 1),
        p1at, p1bt, p1ct, p1.reshape(1, G), P2.T, p2.reshape(1, 1))
    return y, y2, dkl.reshape(())


# speedup vs baseline: 1.6890x; 1.6890x over previous
import functools

import jax
import jax.numpy as jnp
import numpy as np
from jax import lax
from jax.experimental import pallas as pl
from jax.experimental.pallas import tpu as pltpu

B = 1024
D = 64
NUM_GROUPS = 32
G = 8
HID = (D + 2 * D) // 2
Q_STD = float(np.sqrt(2.0 / D))
LOG_SIGMA = float(np.log(Q_STD))


def _sigmoid(x):
    return 1.0 / (1.0 + jnp.exp(-x))


def _dot(a, b):
    return jax.lax.dot(a, b, precision=jax.lax.Precision.HIGHEST,
                       preferred_element_type=jnp.float32)


def _body(ii_ref, gi_ref, std_ref, u_ref, gemb_ref,
          w1t_ref, b1_ref, w2mut_ref, w2sgt_ref, b2mu_ref, b2sg_ref,
          a1mt_ref, a1it_ref, a1_ref, a2t_ref, a2_ref,
          p1at_ref, p1bt_ref, p1ct_ref, p1_ref, p2t_ref, p2_ref,
          item_hbm, y_ref, y2_ref, dkl_ref, item_vmem, sem):
    relu = lambda x: jnp.maximum(x, 0.0)

    def issue(i, _):
        pltpu.make_async_copy(
            item_hbm.at[pl.ds(ii_ref[i], 1)],
            item_vmem.at[pl.ds(i, 1)], sem).start()
        return 0
    lax.fori_loop(0, B, issue, 0, unroll=8)

    U = [u_ref[j] for j in range(G)]
    gsum = U[0]
    for j in range(1, G):
        gsum = gsum + U[j]
    group_z = relu(gsum * (1.0 / G))
    h = relu(_dot(group_z, w1t_ref[...]) + b1_ref[...])
    z_mu = _dot(h, w2mut_ref[...]) + b2mu_ref[...]
    z_sg = 0.1 + 0.9 * _sigmoid(_dot(h, w2sgt_ref[...]) + b2sg_ref[...])

    gi = gi_ref[...]
    gids = lax.broadcasted_iota(jnp.int32, (B, NUM_GROUPS), 1)
    onehot = (gi == gids).astype(jnp.float32)

    gvec = gemb_ref[...]
    t = (2.0 * jnp.log(z_sg) - 2.0 * LOG_SIGMA
         + (Q_STD * Q_STD) / (z_sg * z_sg)
         + ((gvec - z_mu) * (gvec - z_mu)) / (z_sg * z_sg) - 1.0)
    s = jnp.sum(0.5 * t, axis=1, keepdims=True)
    counts = jnp.sum(onehot, axis=0, keepdims=True)
    dkl_ref[...] = _dot(counts, s) * (1.0 / B)

    pltpu.make_async_copy(item_hbm.at[pl.ds(0, B)], item_vmem, sem).wait()
    item = item_vmem[...]
    ipart = _dot(item, a1it_ref[...]) + a1_ref[...]

    att_cols = []
    for j in range(G):
        mp = _dot(U[j], a1mt_ref[...])
        hj = relu(_dot(onehot, mp) + ipart)
        att_cols.append(_dot(hj, a2t_ref[...]))
    att = jnp.concatenate(att_cols, axis=1) + a2_ref[...]
    amax = jnp.max(att, axis=1, keepdims=True)
    aexp = jnp.exp(att - amax)
    w = aexp / jnp.sum(aexp, axis=1, keepdims=True)

    g_att = w[:, 0:1] * _dot(onehot, U[0])
    for j in range(1, G):
        g_att = g_att + w[:, j:j + 1] * _dot(onehot, U[j])

    gp = _dot(onehot, gemb_ref[...])
    zmu_b = _dot(onehot, z_mu)
    zsg_b = _dot(onehot, z_sg)
    std = std_ref[...]
    ge = g_att + gp + Q_STD * std
    ge2 = g_att + zmu_b + zsg_b * std

    def ncf(x):
        h1 = relu(_dot(x * item, p1at_ref[...]) + _dot(x, p1bt_ref[...])
                  + _dot(item, p1ct_ref[...]) + p1_ref[...])
        return _sigmoid(_dot(h1, p2t_ref[...]) + p2_ref[...])

    y_ref[...] = ncf(ge)
    y2_ref[...] = ncf(ge2)


@jax.jit
def _call(ii, gi, std, u8, group_emb,
          w1t, b1, w2mut, w2sgt, b2mu, b2sg,
          a1mt, a1it, a1, a2t, a2, p1at, p1bt, p1ct, p1, p2t, p2, item_emb):
    n_in = 22
    vspec = pl.BlockSpec(memory_space=pltpu.VMEM)
    return pl.pallas_call(
        _body,
        in_specs=[pl.BlockSpec(memory_space=pltpu.SMEM)]
                 + [vspec] * (n_in - 1)
                 + [pl.BlockSpec(memory_space=pl.ANY)],
        out_specs=(vspec, vspec, vspec),
        out_shape=(
            jax.ShapeDtypeStruct((B, 1), jnp.float32),
            jax.ShapeDtypeStruct((B, 1), jnp.float32),
            jax.ShapeDtypeStruct((1, 1), jnp.float32),
        ),
        scratch_shapes=[
            pltpu.VMEM((B, D), jnp.float32),
            pltpu.SemaphoreType.DMA,
        ],
    )(ii, gi, std, u8, group_emb,
      w1t, b1, w2mut, w2sgt, b2mu, b2sg,
      a1mt, a1it, a1, a2t, a2, p1at, p1bt, p1ct, p1, p2t, p2, item_emb)


def kernel(group_inputs, item_inputs, is_training, user_emb, item_emb,
           group_emb, W1, b1, W2, b2, A1, a1, A2, a2, P1, p1, P2, p2):
    gi = group_inputs.astype(jnp.int32).reshape(B, 1)
    ii = item_inputs.astype(jnp.int32)

    u8 = user_emb[:NUM_GROUPS * G].reshape(NUM_GROUPS, G, D).swapaxes(0, 1)
    std = jax.random.normal(jax.random.key(1), (B, D), dtype=jnp.float32)

    w1t = W1.T
    w2mut = W2[:D].T
    w2sgt = W2[D:].T
    a1mt = A1[:, :D].T
    a1it = A1[:, D:].T
    p1at = P1[:, :D].T
    p1bt = P1[:, D:2 * D].T
    p1ct = P1[:, 2 * D:].T

    y, y2, dkl = _call(
        ii, gi, std, u8, group_emb,
        w1t, b1.reshape(1, HID), w2mut, w2sgt,
        b2[:D].reshape(1, D), b2[D:].reshape(1, D),
        a1mt, a1it, a1.reshape(1, 16), A2.T, a2.reshape(1, 1),
        p1at, p1bt, p1ct, p1.reshape(1, G), P2.T, p2.reshape(1, 1),
        item_emb)
    return y, y2, dkl.reshape(())


# speedup vs baseline: 1.6906x; 1.0009x over previous
import functools

import jax
import jax.numpy as jnp
import numpy as np
from jax import lax
from jax.experimental import pallas as pl
from jax.experimental.pallas import tpu as pltpu

B = 1024
D = 64
NUM_GROUPS = 32
G = 8
HID = (D + 2 * D) // 2
Q_STD = float(np.sqrt(2.0 / D))
LOG_SIGMA = float(np.log(Q_STD))


def _sigmoid(x):
    return 1.0 / (1.0 + jnp.exp(-x))


def _dot(a, b):
    return jax.lax.dot(a, b, precision=jax.lax.Precision.HIGHEST,
                       preferred_element_type=jnp.float32)


def _body(ii_ref, gi_ref, std_ref, u_ref, gemb_ref,
          w1t_ref, b1_ref, w2mut_ref, w2sgt_ref, b2mu_ref, b2sg_ref,
          a1mt_ref, a1it_ref, a1_ref, a2t_ref, a2_ref,
          p1at_ref, p1bt_ref, p1ct_ref, p1_ref, p2t_ref, p2_ref,
          item_hbm, y_ref, y2_ref, dkl_ref, item_vmem, sem):
    relu = lambda x: jnp.maximum(x, 0.0)

    NT = 6

    def issue(i, _):
        for k in range(NT):
            r = i * NT + k
            pltpu.make_async_copy(
                item_hbm.at[pl.ds(ii_ref[r], 1)],
                item_vmem.at[pl.ds(r, 1)], sem.at[k]).start()
        return 0
    lax.fori_loop(0, B // NT, issue, 0)
    for k in range(B - NT * (B // NT)):
        r = NT * (B // NT) + k
        pltpu.make_async_copy(
            item_hbm.at[pl.ds(ii_ref[r], 1)],
            item_vmem.at[pl.ds(r, 1)], sem.at[k]).start()

    U = [u_ref[j] for j in range(G)]
    gsum = U[0]
    for j in range(1, G):
        gsum = gsum + U[j]
    group_z = relu(gsum * (1.0 / G))
    h = relu(_dot(group_z, w1t_ref[...]) + b1_ref[...])
    z_mu = _dot(h, w2mut_ref[...]) + b2mu_ref[...]
    z_sg = 0.1 + 0.9 * _sigmoid(_dot(h, w2sgt_ref[...]) + b2sg_ref[...])

    gi = gi_ref[...]
    gids = lax.broadcasted_iota(jnp.int32, (B, NUM_GROUPS), 1)
    onehot = (gi == gids).astype(jnp.float32)

    gvec = gemb_ref[...]
    t = (2.0 * jnp.log(z_sg) - 2.0 * LOG_SIGMA
         + (Q_STD * Q_STD) / (z_sg * z_sg)
         + ((gvec - z_mu) * (gvec - z_mu)) / (z_sg * z_sg) - 1.0)
    s = jnp.sum(0.5 * t, axis=1, keepdims=True)
    counts = jnp.sum(onehot, axis=0, keepdims=True)
    dkl_ref[...] = _dot(counts, s) * (1.0 / B)

    nmain, ntail = B // NT, B - NT * (B // NT)
    for k in range(NT):
        cnt = nmain + (1 if k < ntail else 0)
        pltpu.make_async_copy(
            item_hbm.at[pl.ds(0, cnt)],
            item_vmem.at[pl.ds(0, cnt)], sem.at[k]).wait()
    item = item_vmem[...]
    ipart = _dot(item, a1it_ref[...]) + a1_ref[...]

    att_cols = []
    for j in range(G):
        mp = _dot(U[j], a1mt_ref[...])
        hj = relu(_dot(onehot, mp) + ipart)
        att_cols.append(_dot(hj, a2t_ref[...]))
    att = jnp.concatenate(att_cols, axis=1) + a2_ref[...]
    amax = jnp.max(att, axis=1, keepdims=True)
    aexp = jnp.exp(att - amax)
    w = aexp / jnp.sum(aexp, axis=1, keepdims=True)

    g_att = w[:, 0:1] * _dot(onehot, U[0])
    for j in range(1, G):
        g_att = g_att + w[:, j:j + 1] * _dot(onehot, U[j])

    gp = _dot(onehot, gemb_ref[...])
    zmu_b = _dot(onehot, z_mu)
    zsg_b = _dot(onehot, z_sg)
    std = std_ref[...]
    ge = g_att + gp + Q_STD * std
    ge2 = g_att + zmu_b + zsg_b * std

    def ncf(x):
        h1 = relu(_dot(x * item, p1at_ref[...]) + _dot(x, p1bt_ref[...])
                  + _dot(item, p1ct_ref[...]) + p1_ref[...])
        return _sigmoid(_dot(h1, p2t_ref[...]) + p2_ref[...])

    y_ref[...] = ncf(ge)
    y2_ref[...] = ncf(ge2)


@jax.jit
def _call(ii, gi, std, u8, group_emb,
          w1t, b1, w2mut, w2sgt, b2mu, b2sg,
          a1mt, a1it, a1, a2t, a2, p1at, p1bt, p1ct, p1, p2t, p2, item_emb):
    n_in = 22
    vspec = pl.BlockSpec(memory_space=pltpu.VMEM)
    return pl.pallas_call(
        _body,
        in_specs=[pl.BlockSpec(memory_space=pltpu.SMEM)]
                 + [vspec] * (n_in - 1)
                 + [pl.BlockSpec(memory_space=pl.ANY)],
        out_specs=(vspec, vspec, vspec),
        out_shape=(
            jax.ShapeDtypeStruct((B, 1), jnp.float32),
            jax.ShapeDtypeStruct((B, 1), jnp.float32),
            jax.ShapeDtypeStruct((1, 1), jnp.float32),
        ),
        scratch_shapes=[
            pltpu.VMEM((B, D), jnp.float32),
            pltpu.SemaphoreType.DMA((6,)),
        ],
    )(ii, gi, std, u8, group_emb,
      w1t, b1, w2mut, w2sgt, b2mu, b2sg,
      a1mt, a1it, a1, a2t, a2, p1at, p1bt, p1ct, p1, p2t, p2, item_emb)


def kernel(group_inputs, item_inputs, is_training, user_emb, item_emb,
           group_emb, W1, b1, W2, b2, A1, a1, A2, a2, P1, p1, P2, p2):
    gi = group_inputs.astype(jnp.int32).reshape(B, 1)
    ii = item_inputs.astype(jnp.int32)

    u8 = user_emb[:NUM_GROUPS * G].reshape(NUM_GROUPS, G, D).swapaxes(0, 1)
    std = jax.random.normal(jax.random.key(1), (B, D), dtype=jnp.float32)

    w1t = W1.T
    w2mut = W2[:D].T
    w2sgt = W2[D:].T
    a1mt = A1[:, :D].T
    a1it = A1[:, D:].T
    p1at = P1[:, :D].T
    p1bt = P1[:, D:2 * D].T
    p1ct = P1[:, 2 * D:].T

    y, y2, dkl = _call(
        ii, gi, std, u8, group_emb,
        w1t, b1.reshape(1, HID), w2mut, w2sgt,
        b2[:D].reshape(1, D), b2[D:].reshape(1, D),
        a1mt, a1it, a1.reshape(1, 16), A2.T, a2.reshape(1, 1),
        p1at, p1bt, p1ct, p1.reshape(1, G), P2.T, p2.reshape(1, 1),
        item_emb)
    return y, y2, dkl.reshape(())


# speedup vs baseline: 1.6911x; 1.0003x over previous
import functools

import jax
import jax.numpy as jnp
import numpy as np
from jax import lax
from jax.experimental import pallas as pl
from jax.experimental.pallas import tpu as pltpu

B = 1024
D = 64
NUM_GROUPS = 32
G = 8
HID = (D + 2 * D) // 2
Q_STD = float(np.sqrt(2.0 / D))
LOG_SIGMA = float(np.log(Q_STD))


def _sigmoid(x):
    return 1.0 / (1.0 + jnp.exp(-x))


def _dot(a, b):
    return jax.lax.dot(a, b, precision=jax.lax.Precision.HIGHEST,
                       preferred_element_type=jnp.float32)


def _body(ii_ref, gi_ref, std_ref, u_ref, gemb_ref,
          w1t_ref, b1_ref, w2mut_ref, w2sgt_ref, b2mu_ref, b2sg_ref,
          a1mt_ref, a1it_ref, a1_ref, a2t_ref, a2_ref,
          p1at_ref, p1bt_ref, p1ct_ref, p1_ref, p2t_ref, p2_ref,
          item_hbm, y_ref, y2_ref, dkl_ref, item_vmem, sem):
    relu = lambda x: jnp.maximum(x, 0.0)

    NT = 6

    def issue(i, _):
        for k in range(NT):
            r = i * NT + k
            pltpu.make_async_copy(
                item_hbm.at[pl.ds(ii_ref[r], 1)],
                item_vmem.at[pl.ds(r, 1)], sem.at[k]).start(priority=k & 1)
        return 0
    lax.fori_loop(0, B // NT, issue, 0)
    for k in range(B - NT * (B // NT)):
        r = NT * (B // NT) + k
        pltpu.make_async_copy(
            item_hbm.at[pl.ds(ii_ref[r], 1)],
            item_vmem.at[pl.ds(r, 1)], sem.at[k]).start(priority=k & 1)

    U = [u_ref[j] for j in range(G)]
    gsum = U[0]
    for j in range(1, G):
        gsum = gsum + U[j]
    group_z = relu(gsum * (1.0 / G))
    h = relu(_dot(group_z, w1t_ref[...]) + b1_ref[...])
    z_mu = _dot(h, w2mut_ref[...]) + b2mu_ref[...]
    z_sg = 0.1 + 0.9 * _sigmoid(_dot(h, w2sgt_ref[...]) + b2sg_ref[...])

    gi = gi_ref[...]
    gids = lax.broadcasted_iota(jnp.int32, (B, NUM_GROUPS), 1)
    onehot = (gi == gids).astype(jnp.float32)

    gvec = gemb_ref[...]
    t = (2.0 * jnp.log(z_sg) - 2.0 * LOG_SIGMA
         + (Q_STD * Q_STD) / (z_sg * z_sg)
         + ((gvec - z_mu) * (gvec - z_mu)) / (z_sg * z_sg) - 1.0)
    s = jnp.sum(0.5 * t, axis=1, keepdims=True)
    counts = jnp.sum(onehot, axis=0, keepdims=True)
    dkl_ref[...] = _dot(counts, s) * (1.0 / B)

    nmain, ntail = B // NT, B - NT * (B // NT)
    for k in range(NT):
        cnt = nmain + (1 if k < ntail else 0)
        pltpu.make_async_copy(
            item_hbm.at[pl.ds(0, cnt)],
            item_vmem.at[pl.ds(0, cnt)], sem.at[k]).wait()
    item = item_vmem[...]
    ipart = _dot(item, a1it_ref[...]) + a1_ref[...]

    att_cols = []
    for j in range(G):
        mp = _dot(U[j], a1mt_ref[...])
        hj = relu(_dot(onehot, mp) + ipart)
        att_cols.append(_dot(hj, a2t_ref[...]))
    att = jnp.concatenate(att_cols, axis=1) + a2_ref[...]
    amax = jnp.max(att, axis=1, keepdims=True)
    aexp = jnp.exp(att - amax)
    w = aexp / jnp.sum(aexp, axis=1, keepdims=True)

    g_att = w[:, 0:1] * _dot(onehot, U[0])
    for j in range(1, G):
        g_att = g_att + w[:, j:j + 1] * _dot(onehot, U[j])

    gp = _dot(onehot, gemb_ref[...])
    zmu_b = _dot(onehot, z_mu)
    zsg_b = _dot(onehot, z_sg)
    std = std_ref[...]
    ge = g_att + gp + Q_STD * std
    ge2 = g_att + zmu_b + zsg_b * std

    def ncf(x):
        h1 = relu(_dot(x * item, p1at_ref[...]) + _dot(x, p1bt_ref[...])
                  + _dot(item, p1ct_ref[...]) + p1_ref[...])
        return _sigmoid(_dot(h1, p2t_ref[...]) + p2_ref[...])

    y_ref[...] = ncf(ge)
    y2_ref[...] = ncf(ge2)


@jax.jit
def _call(ii, gi, std, u8, group_emb,
          w1t, b1, w2mut, w2sgt, b2mu, b2sg,
          a1mt, a1it, a1, a2t, a2, p1at, p1bt, p1ct, p1, p2t, p2, item_emb):
    n_in = 22
    vspec = pl.BlockSpec(memory_space=pltpu.VMEM)
    return pl.pallas_call(
        _body,
        in_specs=[pl.BlockSpec(memory_space=pltpu.SMEM)]
                 + [vspec] * (n_in - 1)
                 + [pl.BlockSpec(memory_space=pl.ANY)],
        out_specs=(vspec, vspec, vspec),
        out_shape=(
            jax.ShapeDtypeStruct((B, 1), jnp.float32),
            jax.ShapeDtypeStruct((B, 1), jnp.float32),
            jax.ShapeDtypeStruct((1, 1), jnp.float32),
        ),
        scratch_shapes=[
            pltpu.VMEM((B, D), jnp.float32),
            pltpu.SemaphoreType.DMA((6,)),
        ],
    )(ii, gi, std, u8, group_emb,
      w1t, b1, w2mut, w2sgt, b2mu, b2sg,
      a1mt, a1it, a1, a2t, a2, p1at, p1bt, p1ct, p1, p2t, p2, item_emb)


def kernel(group_inputs, item_inputs, is_training, user_emb, item_emb,
           group_emb, W1, b1, W2, b2, A1, a1, A2, a2, P1, p1, P2, p2):
    gi = group_inputs.astype(jnp.int32).reshape(B, 1)
    ii = item_inputs.astype(jnp.int32)

    u8 = user_emb[:NUM_GROUPS * G].reshape(NUM_GROUPS, G, D).swapaxes(0, 1)
    std = jax.random.normal(jax.random.key(1), (B, D), dtype=jnp.float32)

    w1t = W1.T
    w2mut = W2[:D].T
    w2sgt = W2[D:].T
    a1mt = A1[:, :D].T
    a1it = A1[:, D:].T
    p1at = P1[:, :D].T
    p1bt = P1[:, D:2 * D].T
    p1ct = P1[:, 2 * D:].T

    y, y2, dkl = _call(
        ii, gi, std, u8, group_emb,
        w1t, b1.reshape(1, HID), w2mut, w2sgt,
        b2[:D].reshape(1, D), b2[D:].reshape(1, D),
        a1mt, a1it, a1.reshape(1, 16), A2.T, a2.reshape(1, 1),
        p1at, p1bt, p1ct, p1.reshape(1, G), P2.T, p2.reshape(1, 1),
        item_emb)
    return y, y2, dkl.reshape(())


# speedup vs baseline: 1.6939x; 1.0017x over previous
import functools

import jax
import jax.numpy as jnp
import numpy as np
from jax import lax
from jax.experimental import pallas as pl
from jax.experimental.pallas import tpu as pltpu

B = 1024
D = 64
NUM_GROUPS = 32
G = 8
HID = (D + 2 * D) // 2
Q_STD = float(np.sqrt(2.0 / D))
LOG_SIGMA = float(np.log(Q_STD))


def _sigmoid(x):
    return 1.0 / (1.0 + jnp.exp(-x))


def _dot(a, b):
    return jax.lax.dot(a, b, precision=jax.lax.Precision.HIGHEST,
                       preferred_element_type=jnp.float32)


def _body(ii_ref, gi_ref, std_ref, u_ref, gemb_ref,
          w1t_ref, b1_ref, w2mut_ref, w2sgt_ref, b2mu_ref, b2sg_ref,
          a1mt_ref, a1it_ref, a1_ref, a2t_ref, a2_ref,
          p1at_ref, p1bt_ref, p1ct_ref, p1_ref, p2t_ref, p2_ref,
          item_hbm, y_ref, y2_ref, dkl_ref, item_vmem, sem):
    relu = lambda x: jnp.maximum(x, 0.0)

    NT = 8

    def issue(i, _):
        for k in range(NT):
            r = i * NT + k
            pltpu.make_async_copy(
                item_hbm.at[pl.ds(ii_ref[r], 1)],
                item_vmem.at[pl.ds(r, 1)], sem.at[k]).start(priority=k & 1)
        return 0
    lax.fori_loop(0, B // NT, issue, 0)
    for k in range(B - NT * (B // NT)):
        r = NT * (B // NT) + k
        pltpu.make_async_copy(
            item_hbm.at[pl.ds(ii_ref[r], 1)],
            item_vmem.at[pl.ds(r, 1)], sem.at[k]).start(priority=k & 1)

    U = [u_ref[j] for j in range(G)]
    gsum = U[0]
    for j in range(1, G):
        gsum = gsum + U[j]
    group_z = relu(gsum * (1.0 / G))
    h = relu(_dot(group_z, w1t_ref[...]) + b1_ref[...])
    z_mu = _dot(h, w2mut_ref[...]) + b2mu_ref[...]
    z_sg = 0.1 + 0.9 * _sigmoid(_dot(h, w2sgt_ref[...]) + b2sg_ref[...])

    gi = gi_ref[...]
    gids = lax.broadcasted_iota(jnp.int32, (B, NUM_GROUPS), 1)
    onehot = (gi == gids).astype(jnp.float32)

    gvec = gemb_ref[...]
    t = (2.0 * jnp.log(z_sg) - 2.0 * LOG_SIGMA
         + (Q_STD * Q_STD) / (z_sg * z_sg)
         + ((gvec - z_mu) * (gvec - z_mu)) / (z_sg * z_sg) - 1.0)
    s = jnp.sum(0.5 * t, axis=1, keepdims=True)
    counts = jnp.sum(onehot, axis=0, keepdims=True)
    dkl_ref[...] = _dot(counts, s) * (1.0 / B)

    nmain, ntail = B // NT, B - NT * (B // NT)
    for k in range(NT):
        cnt = nmain + (1 if k < ntail else 0)
        pltpu.make_async_copy(
            item_hbm.at[pl.ds(0, cnt)],
            item_vmem.at[pl.ds(0, cnt)], sem.at[k]).wait()
    item = item_vmem[...]
    ipart = _dot(item, a1it_ref[...]) + a1_ref[...]

    att_cols = []
    for j in range(G):
        mp = _dot(U[j], a1mt_ref[...])
        hj = relu(_dot(onehot, mp) + ipart)
        att_cols.append(_dot(hj, a2t_ref[...]))
    att = jnp.concatenate(att_cols, axis=1) + a2_ref[...]
    amax = jnp.max(att, axis=1, keepdims=True)
    aexp = jnp.exp(att - amax)
    w = aexp / jnp.sum(aexp, axis=1, keepdims=True)

    g_att = w[:, 0:1] * _dot(onehot, U[0])
    for j in range(1, G):
        g_att = g_att + w[:, j:j + 1] * _dot(onehot, U[j])

    gp = _dot(onehot, gemb_ref[...])
    zmu_b = _dot(onehot, z_mu)
    zsg_b = _dot(onehot, z_sg)
    std = std_ref[...]
    ge = g_att + gp + Q_STD * std
    ge2 = g_att + zmu_b + zsg_b * std

    def ncf(x):
        h1 = relu(_dot(x * item, p1at_ref[...]) + _dot(x, p1bt_ref[...])
                  + _dot(item, p1ct_ref[...]) + p1_ref[...])
        return _sigmoid(_dot(h1, p2t_ref[...]) + p2_ref[...])

    y_ref[...] = ncf(ge)
    y2_ref[...] = ncf(ge2)


@jax.jit
def _call(ii, gi, std, u8, group_emb,
          w1t, b1, w2mut, w2sgt, b2mu, b2sg,
          a1mt, a1it, a1, a2t, a2, p1at, p1bt, p1ct, p1, p2t, p2, item_emb):
    n_in = 22
    vspec = pl.BlockSpec(memory_space=pltpu.VMEM)
    return pl.pallas_call(
        _body,
        in_specs=[pl.BlockSpec(memory_space=pltpu.SMEM)]
                 + [vspec] * (n_in - 1)
                 + [pl.BlockSpec(memory_space=pl.ANY)],
        out_specs=(vspec, vspec, vspec),
        out_shape=(
            jax.ShapeDtypeStruct((B, 1), jnp.float32),
            jax.ShapeDtypeStruct((B, 1), jnp.float32),
            jax.ShapeDtypeStruct((1, 1), jnp.float32),
        ),
        scratch_shapes=[
            pltpu.VMEM((B, D), jnp.float32),
            pltpu.SemaphoreType.DMA((8,)),
        ],
    )(ii, gi, std, u8, group_emb,
      w1t, b1, w2mut, w2sgt, b2mu, b2sg,
      a1mt, a1it, a1, a2t, a2, p1at, p1bt, p1ct, p1, p2t, p2, item_emb)


def kernel(group_inputs, item_inputs, is_training, user_emb, item_emb,
           group_emb, W1, b1, W2, b2, A1, a1, A2, a2, P1, p1, P2, p2):
    gi = group_inputs.astype(jnp.int32).reshape(B, 1)
    ii = item_inputs.astype(jnp.int32)

    u8 = user_emb[:NUM_GROUPS * G].reshape(NUM_GROUPS, G, D).swapaxes(0, 1)
    std = jax.random.normal(jax.random.key(1), (B, D), dtype=jnp.float32)

    w1t = W1.T
    w2mut = W2[:D].T
    w2sgt = W2[D:].T
    a1mt = A1[:, :D].T
    a1it = A1[:, D:].T
    p1at = P1[:, :D].T
    p1bt = P1[:, D:2 * D].T
    p1ct = P1[:, 2 * D:].T

    y, y2, dkl = _call(
        ii, gi, std, u8, group_emb,
        w1t, b1.reshape(1, HID), w2mut, w2sgt,
        b2[:D].reshape(1, D), b2[D:].reshape(1, D),
        a1mt, a1it, a1.reshape(1, 16), A2.T, a2.reshape(1, 1),
        p1at, p1bt, p1ct, p1.reshape(1, G), P2.T, p2.reshape(1, 1),
        item_emb)
    return y, y2, dkl.reshape(())
